# Initial kernel scaffold; baseline (speedup 1.0000x reference)
import jax
import jax.numpy as jnp
from jax.experimental import pallas as pl


def kernel(x, edge_attr, edge_index, params):
    raise NotImplementedError("write your pallas kernel here")


# speedup vs baseline: 2.1351x; 2.1351x over previous
import functools

import jax
import jax.numpy as jnp
from jax import lax
from jax.experimental import pallas as pl
from jax.experimental.pallas import tpu as pltpu
from jax.experimental.pallas import tpu_sc as plsc

N = 10000
E = 320000
H = 128
OUT_DIM = 2

NC = 2
NS = 16
NW = NC * NS
EPW = E // NW
CH = 80
NCHUNK = EPW // CH
ROWS_PER_SUB = 632
NPAD = NS * ROWS_PER_SUB

NB = 1000
EB = 2000

_f32 = jnp.float32


def _ln(x, g, b):
    m = jnp.mean(x, axis=-1, keepdims=True)
    v = jnp.mean((x - m) ** 2, axis=-1, keepdims=True)
    return (x - m) * lax.rsqrt(v + 1e-5) * g + b


def _dot(a, b):
    return jnp.dot(a, b, preferred_element_type=_f32)


def _node_enc_body(x, w0, b0, w1, b1, wa, wb, wd, h_o, p_o, q_o, r_o):
    h = _dot(jax.nn.relu(_dot(x[...], w0[...]) + b0[...]), w1[...]) + b1[...]
    h_o[...] = h
    p_o[...] = _dot(h, wa[...])
    q_o[...] = _dot(h, wb[...])
    r_o[...] = _dot(h, wd[...])


def _node_enc(x, w0, b0, w1, b1, wa, wb, wd):
    grid = N // NB
    blk = lambda r, c: pl.BlockSpec((r, c), lambda i: (i, 0))
    wspec = pl.BlockSpec((H, H), lambda i: (0, 0))
    bspec = pl.BlockSpec((1, H), lambda i: (0, 0))
    return pl.pallas_call(
        _node_enc_body,
        grid=(grid,),
        in_specs=[blk(NB, H), wspec, bspec, wspec, bspec, wspec, wspec, wspec],
        out_specs=[blk(NB, H)] * 4,
        out_shape=[jax.ShapeDtypeStruct((N, H), _f32)] * 4,
    )(x, w0, b0, w1, b1, wa, wb, wd)


def _edge_enc_body(ea, w0, b0, w1, b1, e_o):
    e_o[...] = _dot(jax.nn.relu(_dot(ea[...], w0[...]) + b0[...]), w1[...]) + b1[...]


def _edge_enc(ea, w0, b0, w1, b1):
    grid = E // EB
    d_edge = ea.shape[1]
    return pl.pallas_call(
        _edge_enc_body,
        grid=(grid,),
        in_specs=[
            pl.BlockSpec((EB, d_edge), lambda i: (i, 0)),
            pl.BlockSpec((d_edge, H), lambda i: (0, 0)),
            pl.BlockSpec((1, H), lambda i: (0, 0)),
            pl.BlockSpec((H, H), lambda i: (0, 0)),
            pl.BlockSpec((1, H), lambda i: (0, 0)),
        ],
        out_specs=pl.BlockSpec((EB, H), lambda i: (i, 0)),
        out_shape=jax.ShapeDtypeStruct((E, H), _f32),
    )(ea, w0, b0, w1, b1)


def _edge_body(e, g_in, rg, wc, be0, we1, be1, wg, bn0, wn1, bn1, eng, enb,
               e_o, msg_o):
    ev = e[...]
    u = g_in[...] + _dot(ev, wc[...]) + be0[...]
    e_new = _dot(jax.nn.relu(u), we1[...]) + be1[...]
    e_o[...] = _ln(ev + e_new, eng[...], enb[...])
    v = rg[...] + _dot(e_new, wg[...]) + bn0[...]
    msg_o[...] = _dot(jax.nn.relu(v), wn1[...]) + bn1[...]


def _edge_update(e, g_in, rg, wc, be0, we1, be1, wg, bn0, wn1, bn1, eng, enb):
    grid = E // EB
    eblk = pl.BlockSpec((EB, H), lambda i: (i, 0))
    wspec = pl.BlockSpec((H, H), lambda i: (0, 0))
    bspec = pl.BlockSpec((1, H), lambda i: (0, 0))
    return pl.pallas_call(
        _edge_body,
        grid=(grid,),
        in_specs=[eblk, eblk, eblk, wspec, bspec, wspec, bspec, wspec, bspec,
                  wspec, bspec, bspec, bspec],
        out_specs=[eblk, eblk],
        out_shape=[jax.ShapeDtypeStruct((E, H), _f32)] * 2,
    )(e, g_in, rg, wc, be0, we1, be1, wg, bn0, wn1, bn1, eng, enb)


def _node_upd_body(h, s0, s1, g, b, wa, wb, wd, h_o, p_o, q_o, r_o):
    hn = _ln(h[...] + s0[0] + s1[0], g[...], b[...])
    h_o[...] = hn
    p_o[...] = _dot(hn, wa[...])
    q_o[...] = _dot(hn, wb[...])
    r_o[...] = _dot(hn, wd[...])


def _node_update(h, s_parts, g, b, wa, wb, wd):
    grid = N // NB
    blk = pl.BlockSpec((NB, H), lambda i: (i, 0))
    sblk = pl.BlockSpec((1, NB, H), lambda i: (0, i, 0))
    wspec = pl.BlockSpec((H, H), lambda i: (0, 0))
    bspec = pl.BlockSpec((1, H), lambda i: (0, 0))
    return pl.pallas_call(
        _node_upd_body,
        grid=(grid,),
        in_specs=[blk, sblk, sblk, bspec, bspec, wspec, wspec, wspec],
        out_specs=[blk] * 4,
        out_shape=[jax.ShapeDtypeStruct((N, H), _f32)] * 4,
    )(h, s_parts[0:1], s_parts[1:2], g, b, wa, wb, wd)


def _node_dec_body(h, s0, s1, g, b, w0, b0, w1, b1, out_o):
    hn = _ln(h[...] + s0[0] + s1[0], g[...], b[...])
    out_o[...] = _dot(jax.nn.relu(_dot(hn, w0[...]) + b0[...]), w1[...]) + b1[...]


def _node_decode(h, s_parts, g, b, w0, b0, w1, b1):
    grid = N // NB
    blk = pl.BlockSpec((NB, H), lambda i: (i, 0))
    sblk = pl.BlockSpec((1, NB, H), lambda i: (0, i, 0))
    wspec = pl.BlockSpec((H, H), lambda i: (0, 0))
    bspec = pl.BlockSpec((1, H), lambda i: (0, 0))
    return pl.pallas_call(
        _node_dec_body,
        grid=(grid,),
        in_specs=[blk, sblk, sblk, bspec, bspec, wspec, bspec,
                  pl.BlockSpec((H, OUT_DIM), lambda i: (0, 0)),
                  pl.BlockSpec((1, OUT_DIM), lambda i: (0, 0))],
        out_specs=pl.BlockSpec((NB, OUT_DIM), lambda i: (i, 0)),
        out_shape=jax.ShapeDtypeStruct((N, OUT_DIM), _f32),
    )(h, s_parts[0:1], s_parts[1:2], g, b, w0, b0, w1, b1)


def _sc_gather(p, q, r, dst, src):
    mesh = plsc.VectorSubcoreMesh(core_axis_name="c", subcore_axis_name="s")

    @functools.partial(
        pl.kernel,
        out_type=[jax.ShapeDtypeStruct((E, H), _f32)] * 2,
        mesh=mesh,
        scratch_types=[
            pltpu.VMEM((CH,), jnp.int32),
            pltpu.VMEM((CH,), jnp.int32),
            pltpu.VMEM((CH, H), _f32),
            pltpu.VMEM((CH, H), _f32),
            pltpu.SemaphoreType.DMA,
        ],
    )
    def k(p_hbm, q_hbm, r_hbm, dst_hbm, src_hbm, g_hbm, rg_hbm,
          dstv, srcv, bufg, bufr, sem):
        wid = lax.axis_index("s") * NC + lax.axis_index("c")

        def body(j, carry):
            base = wid * EPW + j * CH
            pltpu.sync_copy(dst_hbm.at[pl.ds(base, CH)], dstv)
            pltpu.sync_copy(src_hbm.at[pl.ds(base, CH)], srcv)
            pltpu.async_copy(p_hbm.at[dstv], bufg, sem).wait()
            pltpu.async_copy(q_hbm.at[srcv], bufg, sem, add=True).wait()
            pltpu.async_copy(r_hbm.at[dstv], bufr, sem).wait()
            pltpu.sync_copy(bufg, g_hbm.at[pl.ds(base, CH)])
            pltpu.sync_copy(bufr, rg_hbm.at[pl.ds(base, CH)])
            return carry

        lax.fori_loop(0, NCHUNK, body, 0, unroll=False)

    return k(p, q, r, dst, src)


def _sc_scatter(msg, dst, zeros_nh):
    mesh = plsc.VectorSubcoreMesh(core_axis_name="c", subcore_axis_name="s")

    @functools.partial(
        pl.kernel,
        out_type=jax.ShapeDtypeStruct((NC, NPAD, H), _f32),
        mesh=mesh,
        scratch_types=[
            pltpu.VMEM((CH,), jnp.int32),
            pltpu.VMEM((CH, H), _f32),
            pltpu.VMEM_SHARED((NPAD, H), _f32),
        ],
    )
    def k(msg_hbm, dst_hbm, zero_hbm, out_hbm, dstv, bufm, acc):
        cid = lax.axis_index("c")
        sid = lax.axis_index("s")
        wid = sid * NC + cid
        row0 = sid * ROWS_PER_SUB
        pltpu.sync_copy(zero_hbm.at[pl.ds(row0, ROWS_PER_SUB)],
                        acc.at[pl.ds(row0, ROWS_PER_SUB)])
        plsc.subcore_barrier()

        def body(j, carry):
            base = wid * EPW + j * CH
            pltpu.sync_copy(dst_hbm.at[pl.ds(base, CH)], dstv)
            pltpu.sync_copy(msg_hbm.at[pl.ds(base, CH)], bufm)
            pltpu.sync_copy(bufm, acc.at[dstv], add=True)
            return carry

        lax.fori_loop(0, NCHUNK, body, 0, unroll=False)
        plsc.subcore_barrier()
        pltpu.sync_copy(acc.at[pl.ds(row0, ROWS_PER_SUB)],
                        out_hbm.at[cid, pl.ds(row0, ROWS_PER_SUB)])

    return k(msg, dst, zeros_nh)


def kernel(x, edge_attr, edge_index, params):
    src = edge_index[0]
    dst = edge_index[1]
    zeros_nh = jnp.zeros((NPAD, H), _f32)

    def b2(v):
        return v.reshape(1, -1)

    enc = params["node_enc"]
    eenc = params["edge_enc"]
    gnn = params["gnn"]
    dec = params["decoder"]

    p0 = gnn[0]
    h, P, Q, R = _node_enc(
        x, enc["W0"], b2(enc["b0"]), enc["W1"], b2(enc["b1"]),
        p0["edge_mlp"]["W0"][0:H], p0["edge_mlp"]["W0"][H:2 * H],
        p0["node_mlp"]["W0"][0:H])
    e = _edge_enc(edge_attr, eenc["W0"], b2(eenc["b0"]), eenc["W1"],
                  b2(eenc["b1"]))

    for l in range(2):
        p = gnn[l]
        em, nm = p["edge_mlp"], p["node_mlp"]
        g_in, rg = _sc_gather(P, Q, R, dst, src)
        e, msg = _edge_update(
            e, g_in, rg,
            em["W0"][2 * H:3 * H], b2(em["b0"]), em["W1"], b2(em["b1"]),
            nm["W0"][H:2 * H], b2(nm["b0"]), nm["W1"], b2(nm["b1"]),
            b2(p["en_g"]), b2(p["en_b"]))
        s_parts = _sc_scatter(msg, dst, zeros_nh)
        if l == 0:
            pn = gnn[1]
            h, P, Q, R = _node_update(
                h, s_parts, b2(p["xn_g"]), b2(p["xn_b"]),
                pn["edge_mlp"]["W0"][0:H], pn["edge_mlp"]["W0"][H:2 * H],
                pn["node_mlp"]["W0"][0:H])
        else:
            out = _node_decode(
                h, s_parts, b2(p["xn_g"]), b2(p["xn_b"]),
                dec["W0"], b2(dec["b0"]), dec["W1"], b2(dec["b1"]))
    return out


# speedup vs baseline: 2.3059x; 1.0800x over previous
import functools

import jax
import jax.numpy as jnp
from jax import lax
from jax.experimental import pallas as pl
from jax.experimental.pallas import tpu as pltpu
from jax.experimental.pallas import tpu_sc as plsc

N = 10000
E = 320000
H = 128
OUT_DIM = 2

NC = 2
NS = 16
NW = NC * NS
EPW = E // NW
CH = 80
NCHUNK = EPW // CH
ROWS_PER_SUB = 632
NPAD = NS * ROWS_PER_SUB

NB = 1000
EB = 2000

_f32 = jnp.float32
_bf16 = jnp.bfloat16


def _ln(x, g, b):
    m = jnp.mean(x, axis=-1, keepdims=True)
    v = jnp.mean((x - m) ** 2, axis=-1, keepdims=True)
    return (x - m) * lax.rsqrt(v + 1e-5) * g + b


def _dot(a, b):
    return jnp.dot(a, b, preferred_element_type=_f32)


def _bdot(a, b):
    return jnp.dot(a.astype(_bf16), b, preferred_element_type=_f32)


def _node_enc_body(x, w0, b0, w1, b1, h_o):
    h_o[...] = _dot(jax.nn.relu(_dot(x[...], w0[...]) + b0[...]), w1[...]) + b1[...]


def _node_enc(x, w0, b0, w1, b1):
    grid = N // NB
    blk = pl.BlockSpec((NB, H), lambda i: (i, 0))
    wspec = pl.BlockSpec((H, H), lambda i: (0, 0))
    bspec = pl.BlockSpec((1, H), lambda i: (0, 0))
    return pl.pallas_call(
        _node_enc_body,
        grid=(grid,),
        in_specs=[blk, wspec, bspec, wspec, bspec],
        out_specs=blk,
        out_shape=jax.ShapeDtypeStruct((N, H), _f32),
    )(x, w0, b0, w1, b1)


def _edge_enc_body(ea, w0, b0, w1, b1, e_o):
    t = jax.nn.relu(_dot(ea[...], w0[...]) + b0[...])
    e_o[...] = (_bdot(t, w1[...]) + b1[...]).astype(_bf16)


def _edge_enc(ea, w0, b0, w1, b1):
    grid = E // EB
    d_edge = ea.shape[1]
    return pl.pallas_call(
        _edge_enc_body,
        grid=(grid,),
        in_specs=[
            pl.BlockSpec((EB, d_edge), lambda i: (i, 0)),
            pl.BlockSpec((d_edge, H), lambda i: (0, 0)),
            pl.BlockSpec((1, H), lambda i: (0, 0)),
            pl.BlockSpec((H, H), lambda i: (0, 0)),
            pl.BlockSpec((1, H), lambda i: (0, 0)),
        ],
        out_specs=pl.BlockSpec((EB, H), lambda i: (i, 0)),
        out_shape=jax.ShapeDtypeStruct((E, H), _bf16),
    )(ea, w0, b0, w1, b1)


def _edge_compute(e, hd, hs, wa, wb, wd, wc, be0, we1, be1, wg, bn0, wn1, bn1):
    ev = e[...]
    hdb = hd[...].astype(_bf16)
    hsb = hs[...].astype(_bf16)
    u = (_dot(hdb, wa[...]) + _dot(hsb, wb[...]) + _dot(ev, wc[...])
         + be0[...])
    e_new = _bdot(jax.nn.relu(u), we1[...]) + be1[...]
    v = _dot(hdb, wd[...]) + _bdot(e_new, wg[...]) + bn0[...]
    msg = _bdot(jax.nn.relu(v), wn1[...]) + bn1[...]
    return ev, e_new, msg


def _edge_body_full(e, hd, hs, wa, wb, wd, wc, be0, we1, be1, wg, bn0, wn1,
                    bn1, eng, enb, e_o, msg_o):
    ev, e_new, msg = _edge_compute(e, hd, hs, wa, wb, wd, wc, be0, we1, be1,
                                   wg, bn0, wn1, bn1)
    e_o[...] = _ln(ev.astype(_f32) + e_new, eng[...], enb[...]).astype(_bf16)
    msg_o[...] = msg


def _edge_body_last(e, hd, hs, wa, wb, wd, wc, be0, we1, be1, wg, bn0, wn1,
                    bn1, msg_o):
    _, _, msg = _edge_compute(e, hd, hs, wa, wb, wd, wc, be0, we1, be1,
                              wg, bn0, wn1, bn1)
    msg_o[...] = msg


def _edge_update(e, hd, hs, wa, wb, wd, wc, be0, we1, be1, wg, bn0, wn1, bn1,
                 eng=None, enb=None, last=False):
    grid = E // EB
    eblk = pl.BlockSpec((EB, H), lambda i: (i, 0))
    wspec = pl.BlockSpec((H, H), lambda i: (0, 0))
    bspec = pl.BlockSpec((1, H), lambda i: (0, 0))
    common = [eblk, eblk, eblk, wspec, wspec, wspec, wspec, bspec, wspec,
              bspec, wspec, bspec, wspec, bspec]
    args = (e, hd, hs, wa, wb, wd, wc, be0, we1, be1, wg, bn0, wn1, bn1)
    if last:
        return pl.pallas_call(
            _edge_body_last,
            grid=(grid,),
            in_specs=common,
            out_specs=eblk,
            out_shape=jax.ShapeDtypeStruct((E, H), _f32),
        )(*args)
    return pl.pallas_call(
        _edge_body_full,
        grid=(grid,),
        in_specs=common + [bspec, bspec],
        out_specs=[eblk, eblk],
        out_shape=[jax.ShapeDtypeStruct((E, H), _bf16),
                   jax.ShapeDtypeStruct((E, H), _f32)],
    )(*args, eng, enb)


def _node_upd_body(h, s0, s1, g, b, h_o):
    h_o[...] = _ln(h[...] + s0[0] + s1[0], g[...], b[...])


def _node_update(h, s_parts, g, b):
    grid = N // NB
    blk = pl.BlockSpec((NB, H), lambda i: (i, 0))
    sblk = pl.BlockSpec((1, NB, H), lambda i: (0, i, 0))
    bspec = pl.BlockSpec((1, H), lambda i: (0, 0))
    return pl.pallas_call(
        _node_upd_body,
        grid=(grid,),
        in_specs=[blk, sblk, sblk, bspec, bspec],
        out_specs=blk,
        out_shape=jax.ShapeDtypeStruct((N, H), _f32),
    )(h, s_parts[0:1], s_parts[1:2], g, b)


def _node_dec_body(h, s0, s1, g, b, w0, b0, w1, b1, out_o):
    hn = _ln(h[...] + s0[0] + s1[0], g[...], b[...])
    out_o[...] = _dot(jax.nn.relu(_dot(hn, w0[...]) + b0[...]), w1[...]) + b1[...]


def _node_decode(h, s_parts, g, b, w0, b0, w1, b1):
    grid = N // NB
    blk = pl.BlockSpec((NB, H), lambda i: (i, 0))
    sblk = pl.BlockSpec((1, NB, H), lambda i: (0, i, 0))
    wspec = pl.BlockSpec((H, H), lambda i: (0, 0))
    bspec = pl.BlockSpec((1, H), lambda i: (0, 0))
    return pl.pallas_call(
        _node_dec_body,
        grid=(grid,),
        in_specs=[blk, sblk, sblk, bspec, bspec, wspec, bspec,
                  pl.BlockSpec((H, OUT_DIM), lambda i: (0, 0)),
                  pl.BlockSpec((1, OUT_DIM), lambda i: (0, 0))],
        out_specs=pl.BlockSpec((NB, OUT_DIM), lambda i: (i, 0)),
        out_shape=jax.ShapeDtypeStruct((N, OUT_DIM), _f32),
    )(h, s_parts[0:1], s_parts[1:2], g, b, w0, b0, w1, b1)


def _sc_gather(h, dst, src):
    mesh = plsc.VectorSubcoreMesh(core_axis_name="c", subcore_axis_name="s")

    @functools.partial(
        pl.kernel,
        out_type=[jax.ShapeDtypeStruct((E, H), _f32)] * 2,
        mesh=mesh,
        scratch_types=[
            pltpu.VMEM((CH,), jnp.int32),
            pltpu.VMEM((CH,), jnp.int32),
            pltpu.VMEM((CH, H), _f32),
            pltpu.VMEM((CH, H), _f32),
            pltpu.SemaphoreType.DMA,
        ],
    )
    def k(h_hbm, dst_hbm, src_hbm, hd_hbm, hs_hbm, dstv, srcv, bufd, bufs,
          sem):
        wid = lax.axis_index("s") * NC + lax.axis_index("c")

        def body(j, carry):
            base = wid * EPW + j * CH
            pltpu.sync_copy(dst_hbm.at[pl.ds(base, CH)], dstv)
            pltpu.sync_copy(src_hbm.at[pl.ds(base, CH)], srcv)
            pltpu.async_copy(h_hbm.at[dstv], bufd, sem).wait()
            pltpu.async_copy(h_hbm.at[srcv], bufs, sem).wait()
            pltpu.sync_copy(bufd, hd_hbm.at[pl.ds(base, CH)])
            pltpu.sync_copy(bufs, hs_hbm.at[pl.ds(base, CH)])
            return carry

        lax.fori_loop(0, NCHUNK, body, 0, unroll=False)

    return k(h, dst, src)


def _sc_scatter(msg, dst, zeros_nh):
    mesh = plsc.VectorSubcoreMesh(core_axis_name="c", subcore_axis_name="s")

    @functools.partial(
        pl.kernel,
        out_type=jax.ShapeDtypeStruct((NC, NPAD, H), _f32),
        mesh=mesh,
        scratch_types=[
            pltpu.VMEM((CH,), jnp.int32),
            pltpu.VMEM((CH, H), _f32),
            pltpu.VMEM_SHARED((NPAD, H), _f32),
        ],
    )
    def k(msg_hbm, dst_hbm, zero_hbm, out_hbm, dstv, bufm, acc):
        cid = lax.axis_index("c")
        sid = lax.axis_index("s")
        wid = sid * NC + cid
        row0 = sid * ROWS_PER_SUB
        pltpu.sync_copy(zero_hbm.at[pl.ds(row0, ROWS_PER_SUB)],
                        acc.at[pl.ds(row0, ROWS_PER_SUB)])
        plsc.subcore_barrier()

        def body(j, carry):
            base = wid * EPW + j * CH
            pltpu.sync_copy(dst_hbm.at[pl.ds(base, CH)], dstv)
            pltpu.sync_copy(msg_hbm.at[pl.ds(base, CH)], bufm)
            pltpu.sync_copy(bufm, acc.at[dstv], add=True)
            return carry

        lax.fori_loop(0, NCHUNK, body, 0, unroll=False)
        plsc.subcore_barrier()
        pltpu.sync_copy(acc.at[pl.ds(row0, ROWS_PER_SUB)],
                        out_hbm.at[cid, pl.ds(row0, ROWS_PER_SUB)])

    return k(msg, dst, zeros_nh)


def kernel(x, edge_attr, edge_index, params):
    src = edge_index[0]
    dst = edge_index[1]
    zeros_nh = jnp.zeros((NPAD, H), _f32)

    def b2(v):
        return v.reshape(1, -1)

    def wb(w):
        return w.astype(_bf16)

    enc = params["node_enc"]
    eenc = params["edge_enc"]
    gnn = params["gnn"]
    dec = params["decoder"]

    h = _node_enc(x, enc["W0"], b2(enc["b0"]), enc["W1"], b2(enc["b1"]))
    e = _edge_enc(edge_attr, eenc["W0"], b2(eenc["b0"]), wb(eenc["W1"]),
                  b2(eenc["b1"]))

    for l in range(2):
        p = gnn[l]
        em, nm = p["edge_mlp"], p["node_mlp"]
        hd, hs = _sc_gather(h, dst, src)
        args = (e, hd, hs,
                wb(em["W0"][0:H]), wb(em["W0"][H:2 * H]),
                wb(nm["W0"][0:H]), wb(em["W0"][2 * H:3 * H]),
                b2(em["b0"]), wb(em["W1"]), b2(em["b1"]),
                wb(nm["W0"][H:2 * H]), b2(nm["b0"]),
                wb(nm["W1"]), b2(nm["b1"]))
        if l == 0:
            e, msg = _edge_update(*args, b2(p["en_g"]), b2(p["en_b"]))
        else:
            msg = _edge_update(*args, last=True)
        s_parts = _sc_scatter(msg, dst, zeros_nh)
        if l == 0:
            h = _node_update(h, s_parts, b2(p["xn_g"]), b2(p["xn_b"]))
        else:
            out = _node_decode(h, s_parts, b2(p["xn_g"]), b2(p["xn_b"]),
                               dec["W0"], b2(dec["b0"]), dec["W1"],
                               b2(dec["b1"]))
    return out


# speedup vs baseline: 3.3476x; 1.4518x over previous
import functools

import jax
import jax.numpy as jnp
from jax import lax
from jax.experimental import pallas as pl
from jax.experimental.pallas import tpu as pltpu
from jax.experimental.pallas import tpu_sc as plsc

N = 10000
E = 320000
H = 128
OUT_DIM = 2

NC = 2
NS = 16
NW = NC * NS
EPW = E // NW
CH = 80
NCHUNK = EPW // CH
ROWS_PER_SUB = 632
NPAD = NS * ROWS_PER_SUB

NB = 1000
EB = 2000

_f32 = jnp.float32
_bf16 = jnp.bfloat16


def _ln(x, g, b):
    m = jnp.mean(x, axis=-1, keepdims=True)
    v = jnp.mean((x - m) ** 2, axis=-1, keepdims=True)
    return (x - m) * lax.rsqrt(v + 1e-5) * g + b


def _dot(a, b):
    return jnp.dot(a, b, preferred_element_type=_f32)


def _bdot(a, b):
    return jnp.dot(a.astype(_bf16), b, preferred_element_type=_f32)


def _node_enc_body(x, w0, b0, w1, b1, h_o):
    h_o[...] = _dot(jax.nn.relu(_dot(x[...], w0[...]) + b0[...]), w1[...]) + b1[...]


def _node_enc(x, w0, b0, w1, b1):
    grid = N // NB
    blk = pl.BlockSpec((NB, H), lambda i: (i, 0))
    wspec = pl.BlockSpec((H, H), lambda i: (0, 0))
    bspec = pl.BlockSpec((1, H), lambda i: (0, 0))
    return pl.pallas_call(
        _node_enc_body,
        grid=(grid,),
        in_specs=[blk, wspec, bspec, wspec, bspec],
        out_specs=blk,
        out_shape=jax.ShapeDtypeStruct((N, H), _f32),
    )(x, w0, b0, w1, b1)


def _edge_enc_body(ea, w0, b0, w1, b1, e_o):
    t = jax.nn.relu(_dot(ea[...], w0[...]) + b0[...])
    e_o[...] = (_bdot(t, w1[...]) + b1[...]).astype(_bf16)


def _edge_enc(ea, w0, b0, w1, b1):
    grid = E // EB
    d_edge = ea.shape[1]
    return pl.pallas_call(
        _edge_enc_body,
        grid=(grid,),
        in_specs=[
            pl.BlockSpec((EB, d_edge), lambda i: (i, 0)),
            pl.BlockSpec((d_edge, H), lambda i: (0, 0)),
            pl.BlockSpec((1, H), lambda i: (0, 0)),
            pl.BlockSpec((H, H), lambda i: (0, 0)),
            pl.BlockSpec((1, H), lambda i: (0, 0)),
        ],
        out_specs=pl.BlockSpec((EB, H), lambda i: (i, 0)),
        out_shape=jax.ShapeDtypeStruct((E, H), _bf16),
    )(ea, w0, b0, w1, b1)


def _edge_compute(e, hd, hs, wa, wb, wd, wc, be0, we1, be1, wg, bn0, wn1, bn1):
    ev = e[...]
    hdb = hd[...].astype(_bf16)
    hsb = hs[...].astype(_bf16)
    u = (_dot(hdb, wa[...]) + _dot(hsb, wb[...]) + _dot(ev, wc[...])
         + be0[...])
    e_new = _bdot(jax.nn.relu(u), we1[...]) + be1[...]
    v = _dot(hdb, wd[...]) + _bdot(e_new, wg[...]) + bn0[...]
    msg = _bdot(jax.nn.relu(v), wn1[...]) + bn1[...]
    return ev, e_new, msg


def _edge_body_full(e, hd, hs, wa, wb, wd, wc, be0, we1, be1, wg, bn0, wn1,
                    bn1, eng, enb, e_o, msg_o):
    ev, e_new, msg = _edge_compute(e, hd, hs, wa, wb, wd, wc, be0, we1, be1,
                                   wg, bn0, wn1, bn1)
    e_o[...] = _ln(ev.astype(_f32) + e_new, eng[...], enb[...]).astype(_bf16)
    msg_o[...] = msg


def _edge_body_last(e, hd, hs, wa, wb, wd, wc, be0, we1, be1, wg, bn0, wn1,
                    bn1, msg_o):
    _, _, msg = _edge_compute(e, hd, hs, wa, wb, wd, wc, be0, we1, be1,
                              wg, bn0, wn1, bn1)
    msg_o[...] = msg


def _edge_update(e, hd, hs, wa, wb, wd, wc, be0, we1, be1, wg, bn0, wn1, bn1,
                 eng=None, enb=None, last=False):
    grid = E // EB
    eblk = pl.BlockSpec((EB, H), lambda i: (i, 0))
    wspec = pl.BlockSpec((H, H), lambda i: (0, 0))
    bspec = pl.BlockSpec((1, H), lambda i: (0, 0))
    common = [eblk, eblk, eblk, wspec, wspec, wspec, wspec, bspec, wspec,
              bspec, wspec, bspec, wspec, bspec]
    args = (e, hd, hs, wa, wb, wd, wc, be0, we1, be1, wg, bn0, wn1, bn1)
    if last:
        return pl.pallas_call(
            _edge_body_last,
            grid=(grid,),
            in_specs=common,
            out_specs=eblk,
            out_shape=jax.ShapeDtypeStruct((E, H), _f32),
        )(*args)
    return pl.pallas_call(
        _edge_body_full,
        grid=(grid,),
        in_specs=common + [bspec, bspec],
        out_specs=[eblk, eblk],
        out_shape=[jax.ShapeDtypeStruct((E, H), _bf16),
                   jax.ShapeDtypeStruct((E, H), _f32)],
    )(*args, eng, enb)


def _node_upd_body(h, s0, s1, g, b, h_o):
    h_o[...] = _ln(h[...] + s0[0] + s1[0], g[...], b[...])


def _node_update(h, s_parts, g, b):
    grid = N // NB
    blk = pl.BlockSpec((NB, H), lambda i: (i, 0))
    sblk = pl.BlockSpec((1, NB, H), lambda i: (0, i, 0))
    bspec = pl.BlockSpec((1, H), lambda i: (0, 0))
    return pl.pallas_call(
        _node_upd_body,
        grid=(grid,),
        in_specs=[blk, sblk, sblk, bspec, bspec],
        out_specs=blk,
        out_shape=jax.ShapeDtypeStruct((N, H), _f32),
    )(h, s_parts[0:1], s_parts[1:2], g, b)


def _node_dec_body(h, s0, s1, g, b, w0, b0, w1, b1, out_o):
    hn = _ln(h[...] + s0[0] + s1[0], g[...], b[...])
    out_o[...] = _dot(jax.nn.relu(_dot(hn, w0[...]) + b0[...]), w1[...]) + b1[...]


def _node_decode(h, s_parts, g, b, w0, b0, w1, b1):
    grid = N // NB
    blk = pl.BlockSpec((NB, H), lambda i: (i, 0))
    sblk = pl.BlockSpec((1, NB, H), lambda i: (0, i, 0))
    wspec = pl.BlockSpec((H, H), lambda i: (0, 0))
    bspec = pl.BlockSpec((1, H), lambda i: (0, 0))
    return pl.pallas_call(
        _node_dec_body,
        grid=(grid,),
        in_specs=[blk, sblk, sblk, bspec, bspec, wspec, bspec,
                  pl.BlockSpec((H, OUT_DIM), lambda i: (0, 0)),
                  pl.BlockSpec((1, OUT_DIM), lambda i: (0, 0))],
        out_specs=pl.BlockSpec((NB, OUT_DIM), lambda i: (i, 0)),
        out_shape=jax.ShapeDtypeStruct((N, OUT_DIM), _f32),
    )(h, s_parts[0:1], s_parts[1:2], g, b, w0, b0, w1, b1)


RING = 4


def _sc_gather(h, dst3, src3):
    mesh = plsc.VectorSubcoreMesh(core_axis_name="c", subcore_axis_name="s")

    @functools.partial(
        pl.kernel,
        out_type=[jax.ShapeDtypeStruct((E, H), _f32)] * 2,
        mesh=mesh,
        scratch_types=[
            pltpu.VMEM((NCHUNK, CH), jnp.int32),
            pltpu.VMEM((NCHUNK, CH), jnp.int32),
            pltpu.VMEM((RING * CH, H), _f32),
            pltpu.VMEM((RING * CH, H), _f32),
            [pltpu.SemaphoreType.DMA] * RING,
            [pltpu.SemaphoreType.DMA] * RING,
        ],
    )
    def k(h_hbm, dst_hbm, src_hbm, hd_hbm, hs_hbm, dsti, srci, bufd, bufs,
          semg, semw):
        wid = lax.axis_index("s") * NC + lax.axis_index("c")
        pltpu.sync_copy(dst_hbm.at[wid], dsti)
        pltpu.sync_copy(src_hbm.at[wid], srci)

        def start_gather(j, s):
            off = s * CH
            pltpu.async_copy(h_hbm.at[dsti.at[j]], bufd.at[pl.ds(off, CH)],
                             semg[s])
            pltpu.async_copy(h_hbm.at[srci.at[j]], bufs.at[pl.ds(off, CH)],
                             semg[s])

        def drain_gather(s):
            off = s * CH
            pltpu.make_async_copy(hd_hbm.at[pl.ds(0, CH)],
                                  bufd.at[pl.ds(off, CH)], semg[s]).wait()
            pltpu.make_async_copy(hd_hbm.at[pl.ds(0, CH)],
                                  bufs.at[pl.ds(off, CH)], semg[s]).wait()

        def start_writeback(j, s):
            off = s * CH
            base = wid * EPW + j * CH
            pltpu.async_copy(bufd.at[pl.ds(off, CH)],
                             hd_hbm.at[pl.ds(base, CH)], semw[s])
            pltpu.async_copy(bufs.at[pl.ds(off, CH)],
                             hs_hbm.at[pl.ds(base, CH)], semw[s])

        def drain_writeback(s):
            off = s * CH
            pltpu.make_async_copy(hd_hbm.at[pl.ds(0, CH)],
                                  bufd.at[pl.ds(off, CH)], semw[s]).wait()
            pltpu.make_async_copy(hd_hbm.at[pl.ds(0, CH)],
                                  bufs.at[pl.ds(off, CH)], semw[s]).wait()

        def stage(j, s):
            s2 = (s + 2) % RING

            @pl.when(j <= NCHUNK - 3)
            def _():
                @pl.when(j >= 2)
                def _():
                    drain_writeback(s2)
                start_gather(j + 2, s2)

            drain_gather(s)
            start_writeback(j, s)

        start_gather(0, 0)
        start_gather(1, 1)

        def body(jj, carry):
            for s in range(RING):
                stage(jj * RING + s, s)
            return carry

        lax.fori_loop(0, NCHUNK // RING, body, 0, unroll=False)
        stage(NCHUNK - 1, (NCHUNK - 1) % RING)
        for jt in range(NCHUNK - RING, NCHUNK):
            drain_writeback(jt % RING)

    return k(h, dst3, src3)


def _sc_scatter(msg, dst, zeros_nh):
    mesh = plsc.VectorSubcoreMesh(core_axis_name="c", subcore_axis_name="s")

    @functools.partial(
        pl.kernel,
        out_type=jax.ShapeDtypeStruct((NC, NPAD, H), _f32),
        mesh=mesh,
        scratch_types=[
            pltpu.VMEM((RING, CH), jnp.int32),
            pltpu.VMEM((RING * CH, H), _f32),
            pltpu.VMEM_SHARED((NPAD, H), _f32),
            [pltpu.SemaphoreType.DMA] * RING,
            [pltpu.SemaphoreType.DMA] * RING,
        ],
    )
    def k(msg_hbm, dst_hbm, zero_hbm, out_hbm, idxr, bufm, acc, seml, sema):
        cid = lax.axis_index("c")
        sid = lax.axis_index("s")
        wid = sid * NC + cid
        row0 = sid * ROWS_PER_SUB
        pltpu.sync_copy(zero_hbm.at[pl.ds(row0, ROWS_PER_SUB)],
                        acc.at[pl.ds(row0, ROWS_PER_SUB)])
        plsc.subcore_barrier()

        def start_load(j, s):
            base = wid * EPW + j * CH
            pltpu.async_copy(msg_hbm.at[pl.ds(base, CH)],
                             bufm.at[pl.ds(s * CH, CH)], seml[s])
            pltpu.async_copy(dst_hbm.at[wid, j], idxr.at[s], seml[s])

        def drain_load(s):
            pltpu.make_async_copy(msg_hbm.at[pl.ds(0, CH)],
                                  bufm.at[pl.ds(s * CH, CH)], seml[s]).wait()
            pltpu.make_async_copy(dst_hbm.at[0, 0], idxr.at[s],
                                  seml[s]).wait()

        def drain_add(s):
            pltpu.make_async_copy(msg_hbm.at[pl.ds(0, CH)],
                                  bufm.at[pl.ds(s * CH, CH)], sema[s]).wait()

        def stage(j, s):
            s2 = (s + 2) % RING

            @pl.when(j <= NCHUNK - 3)
            def _():
                @pl.when(j >= 2)
                def _():
                    drain_add(s2)
                start_load(j + 2, s2)

            drain_load(s)
            pltpu.async_copy(bufm.at[pl.ds(s * CH, CH)], acc.at[idxr.at[s]],
                             sema[s], add=True)

        start_load(0, 0)
        start_load(1, 1)

        def body(jj, carry):
            for s in range(RING):
                stage(jj * RING + s, s)
            return carry

        lax.fori_loop(0, NCHUNK // RING, body, 0, unroll=False)
        stage(NCHUNK - 1, (NCHUNK - 1) % RING)
        for jt in range(NCHUNK - RING, NCHUNK):
            drain_add(jt % RING)
        plsc.subcore_barrier()
        pltpu.sync_copy(acc.at[pl.ds(row0, ROWS_PER_SUB)],
                        out_hbm.at[cid, pl.ds(row0, ROWS_PER_SUB)])

    return k(msg, dst, zeros_nh)


def kernel(x, edge_attr, edge_index, params):
    src = edge_index[0]
    dst = edge_index[1]
    dst3 = dst.reshape(NW, NCHUNK, CH)
    src3 = src.reshape(NW, NCHUNK, CH)
    zeros_nh = jnp.zeros((NPAD, H), _f32)

    def b2(v):
        return v.reshape(1, -1)

    def wb(w):
        return w.astype(_bf16)

    enc = params["node_enc"]
    eenc = params["edge_enc"]
    gnn = params["gnn"]
    dec = params["decoder"]

    h = _node_enc(x, enc["W0"], b2(enc["b0"]), enc["W1"], b2(enc["b1"]))
    e = _edge_enc(edge_attr, eenc["W0"], b2(eenc["b0"]), wb(eenc["W1"]),
                  b2(eenc["b1"]))

    for l in range(2):
        p = gnn[l]
        em, nm = p["edge_mlp"], p["node_mlp"]
        hd, hs = _sc_gather(h, dst3, src3)
        args = (e, hd, hs,
                wb(em["W0"][0:H]), wb(em["W0"][H:2 * H]),
                wb(nm["W0"][0:H]), wb(em["W0"][2 * H:3 * H]),
                b2(em["b0"]), wb(em["W1"]), b2(em["b1"]),
                wb(nm["W0"][H:2 * H]), b2(nm["b0"]),
                wb(nm["W1"]), b2(nm["b1"]))
        if l == 0:
            e, msg = _edge_update(*args, b2(p["en_g"]), b2(p["en_b"]))
        else:
            msg = _edge_update(*args, last=True)
        s_parts = _sc_scatter(msg, dst3, zeros_nh)
        if l == 0:
            h = _node_update(h, s_parts, b2(p["xn_g"]), b2(p["xn_b"]))
        else:
            out = _node_decode(h, s_parts, b2(p["xn_g"]), b2(p["xn_b"]),
                               dec["W0"], b2(dec["b0"]), dec["W1"],
                               b2(dec["b1"]))
    return out


# speedup vs baseline: 3.5948x; 1.0738x over previous
import functools

import jax
import jax.numpy as jnp
from jax import lax
from jax.experimental import pallas as pl
from jax.experimental.pallas import tpu as pltpu
from jax.experimental.pallas import tpu_sc as plsc

N = 10000
E = 320000
H = 128
OUT_DIM = 2

NC = 2
NS = 16
NW = NC * NS
CH = 80
ROWS_PER_SUB = 632
NPAD = NS * ROWS_PER_SUB

K = 5
ECH = E // K
EPW = ECH // NW
NCHUNK = EPW // CH

NB = 1000
EB = 2000

_f32 = jnp.float32
_bf16 = jnp.bfloat16


def _ln(x, g, b):
    m = jnp.mean(x, axis=-1, keepdims=True)
    v = jnp.mean((x - m) ** 2, axis=-1, keepdims=True)
    return (x - m) * lax.rsqrt(v + 1e-5) * g + b


def _dot(a, b):
    return jnp.dot(a, b, preferred_element_type=_f32)


def _bdot(a, b):
    return jnp.dot(a.astype(_bf16), b, preferred_element_type=_f32)


def _node_enc_body(x, w0, b0, w1, b1, h_o):
    h_o[...] = _dot(jax.nn.relu(_dot(x[...], w0[...]) + b0[...]), w1[...]) + b1[...]


def _node_enc(x, w0, b0, w1, b1):
    grid = N // NB
    blk = pl.BlockSpec((NB, H), lambda i: (i, 0))
    wspec = pl.BlockSpec((H, H), lambda i: (0, 0))
    bspec = pl.BlockSpec((1, H), lambda i: (0, 0))
    return pl.pallas_call(
        _node_enc_body,
        grid=(grid,),
        in_specs=[blk, wspec, bspec, wspec, bspec],
        out_specs=blk,
        out_shape=jax.ShapeDtypeStruct((N, H), _f32),
    )(x, w0, b0, w1, b1)


def _edge_enc_body(ea, w0, b0, w1, b1, e_o):
    t = jax.nn.relu(_dot(ea[...], w0[...]) + b0[...])
    e_o[...] = (_bdot(t, w1[...]) + b1[...]).astype(_bf16)


def _edge_enc(ea, w0, b0, w1, b1):
    grid = ECH // EB
    d_edge = ea.shape[1]
    return pl.pallas_call(
        _edge_enc_body,
        grid=(grid,),
        in_specs=[
            pl.BlockSpec((EB, d_edge), lambda i: (i, 0)),
            pl.BlockSpec((d_edge, H), lambda i: (0, 0)),
            pl.BlockSpec((1, H), lambda i: (0, 0)),
            pl.BlockSpec((H, H), lambda i: (0, 0)),
            pl.BlockSpec((1, H), lambda i: (0, 0)),
        ],
        out_specs=pl.BlockSpec((EB, H), lambda i: (i, 0)),
        out_shape=jax.ShapeDtypeStruct((ECH, H), _bf16),
    )(ea, w0, b0, w1, b1)


def _edge_compute(e, hd, hs, wa, wb, wd, wc, be0, we1, be1, wg, bn0, wn1, bn1):
    ev = e[...]
    hdb = hd[...].astype(_bf16)
    hsb = hs[...].astype(_bf16)
    u = (_dot(hdb, wa[...]) + _dot(hsb, wb[...]) + _dot(ev, wc[...])
         + be0[...])
    e_new = _bdot(jax.nn.relu(u), we1[...]) + be1[...]
    v = _dot(hdb, wd[...]) + _bdot(e_new, wg[...]) + bn0[...]
    msg = _bdot(jax.nn.relu(v), wn1[...]) + bn1[...]
    return ev, e_new, msg


def _edge_body_full(e, hd, hs, wa, wb, wd, wc, be0, we1, be1, wg, bn0, wn1,
                    bn1, eng, enb, e_o, msg_o):
    ev, e_new, msg = _edge_compute(e, hd, hs, wa, wb, wd, wc, be0, we1, be1,
                                   wg, bn0, wn1, bn1)
    e_o[...] = _ln(ev.astype(_f32) + e_new, eng[...], enb[...]).astype(_bf16)
    msg_o[...] = msg


def _edge_body_last(e, hd, hs, wa, wb, wd, wc, be0, we1, be1, wg, bn0, wn1,
                    bn1, msg_o):
    _, _, msg = _edge_compute(e, hd, hs, wa, wb, wd, wc, be0, we1, be1,
                              wg, bn0, wn1, bn1)
    msg_o[...] = msg


def _edge_update(e, hd, hs, wa, wb, wd, wc, be0, we1, be1, wg, bn0, wn1, bn1,
                 eng=None, enb=None, last=False):
    grid = ECH // EB
    eblk = pl.BlockSpec((EB, H), lambda i: (i, 0))
    wspec = pl.BlockSpec((H, H), lambda i: (0, 0))
    bspec = pl.BlockSpec((1, H), lambda i: (0, 0))
    common = [eblk, eblk, eblk, wspec, wspec, wspec, wspec, bspec, wspec,
              bspec, wspec, bspec, wspec, bspec]
    args = (e, hd, hs, wa, wb, wd, wc, be0, we1, be1, wg, bn0, wn1, bn1)
    if last:
        return pl.pallas_call(
            _edge_body_last,
            grid=(grid,),
            in_specs=common,
            out_specs=eblk,
            out_shape=jax.ShapeDtypeStruct((ECH, H), _f32),
        )(*args)
    return pl.pallas_call(
        _edge_body_full,
        grid=(grid,),
        in_specs=common + [bspec, bspec],
        out_specs=[eblk, eblk],
        out_shape=[jax.ShapeDtypeStruct((ECH, H), _bf16),
                   jax.ShapeDtypeStruct((ECH, H), _f32)],
    )(*args, eng, enb)


def _node_upd_body(h, s0, s1, g, b, h_o):
    h_o[...] = _ln(h[...] + s0[0] + s1[0], g[...], b[...])


def _node_update(h, s_parts, g, b):
    grid = N // NB
    blk = pl.BlockSpec((NB, H), lambda i: (i, 0))
    sblk = pl.BlockSpec((1, NB, H), lambda i: (0, i, 0))
    bspec = pl.BlockSpec((1, H), lambda i: (0, 0))
    return pl.pallas_call(
        _node_upd_body,
        grid=(grid,),
        in_specs=[blk, sblk, sblk, bspec, bspec],
        out_specs=blk,
        out_shape=jax.ShapeDtypeStruct((N, H), _f32),
    )(h, s_parts[0:1], s_parts[1:2], g, b)


def _node_dec_body(h, s0, s1, g, b, w0, b0, w1, b1, out_o):
    hn = _ln(h[...] + s0[0] + s1[0], g[...], b[...])
    out_o[...] = _dot(jax.nn.relu(_dot(hn, w0[...]) + b0[...]), w1[...]) + b1[...]


def _node_decode(h, s_parts, g, b, w0, b0, w1, b1):
    grid = N // NB
    blk = pl.BlockSpec((NB, H), lambda i: (i, 0))
    sblk = pl.BlockSpec((1, NB, H), lambda i: (0, i, 0))
    wspec = pl.BlockSpec((H, H), lambda i: (0, 0))
    bspec = pl.BlockSpec((1, H), lambda i: (0, 0))
    return pl.pallas_call(
        _node_dec_body,
        grid=(grid,),
        in_specs=[blk, sblk, sblk, bspec, bspec, wspec, bspec,
                  pl.BlockSpec((H, OUT_DIM), lambda i: (0, 0)),
                  pl.BlockSpec((1, OUT_DIM), lambda i: (0, 0))],
        out_specs=pl.BlockSpec((NB, OUT_DIM), lambda i: (i, 0)),
        out_shape=jax.ShapeDtypeStruct((N, OUT_DIM), _f32),
    )(h, s_parts[0:1], s_parts[1:2], g, b, w0, b0, w1, b1)


RING = 4


def _sc_gather(h, dst3, src3):
    mesh = plsc.VectorSubcoreMesh(core_axis_name="c", subcore_axis_name="s")

    @functools.partial(
        pl.kernel,
        out_type=[jax.ShapeDtypeStruct((ECH, H), _f32)] * 2,
        mesh=mesh,
        scratch_types=[
            pltpu.VMEM((NCHUNK, CH), jnp.int32),
            pltpu.VMEM((NCHUNK, CH), jnp.int32),
            pltpu.VMEM((RING * CH, H), _f32),
            pltpu.VMEM((RING * CH, H), _f32),
            [pltpu.SemaphoreType.DMA] * RING,
            [pltpu.SemaphoreType.DMA] * RING,
        ],
    )
    def k(h_hbm, dst_hbm, src_hbm, hd_hbm, hs_hbm, dsti, srci, bufd, bufs,
          semg, semw):
        wid = lax.axis_index("s") * NC + lax.axis_index("c")
        pltpu.sync_copy(dst_hbm.at[wid], dsti)
        pltpu.sync_copy(src_hbm.at[wid], srci)

        def start_gather(j, s):
            off = s * CH
            pltpu.async_copy(h_hbm.at[dsti.at[j]], bufd.at[pl.ds(off, CH)],
                             semg[s])
            pltpu.async_copy(h_hbm.at[srci.at[j]], bufs.at[pl.ds(off, CH)],
                             semg[s])

        def drain_gather(s):
            off = s * CH
            pltpu.make_async_copy(hd_hbm.at[pl.ds(0, CH)],
                                  bufd.at[pl.ds(off, CH)], semg[s]).wait()
            pltpu.make_async_copy(hd_hbm.at[pl.ds(0, CH)],
                                  bufs.at[pl.ds(off, CH)], semg[s]).wait()

        def start_writeback(j, s):
            off = s * CH
            base = wid * EPW + j * CH
            pltpu.async_copy(bufd.at[pl.ds(off, CH)],
                             hd_hbm.at[pl.ds(base, CH)], semw[s])
            pltpu.async_copy(bufs.at[pl.ds(off, CH)],
                             hs_hbm.at[pl.ds(base, CH)], semw[s])

        def drain_writeback(s):
            off = s * CH
            pltpu.make_async_copy(hd_hbm.at[pl.ds(0, CH)],
                                  bufd.at[pl.ds(off, CH)], semw[s]).wait()
            pltpu.make_async_copy(hd_hbm.at[pl.ds(0, CH)],
                                  bufs.at[pl.ds(off, CH)], semw[s]).wait()

        def stage(j, s):
            s2 = (s + 2) % RING

            @pl.when(j <= NCHUNK - 3)
            def _():
                @pl.when(j >= 2)
                def _():
                    drain_writeback(s2)
                start_gather(j + 2, s2)

            drain_gather(s)
            start_writeback(j, s)

        start_gather(0, 0)
        start_gather(1, 1)

        def body(jj, carry):
            for s in range(RING):
                stage(jj * RING + s, s)
            return carry

        lax.fori_loop(0, NCHUNK // RING, body, 0, unroll=False)
        stage(NCHUNK - 1, (NCHUNK - 1) % RING)
        for jt in range(NCHUNK - RING, NCHUNK):
            drain_writeback(jt % RING)

    return k(h, dst3, src3)


def _sc_scatter(msg, dst, init):
    mesh = plsc.VectorSubcoreMesh(core_axis_name="c", subcore_axis_name="s")

    @functools.partial(
        pl.kernel,
        out_type=jax.ShapeDtypeStruct((NC, NPAD, H), _f32),
        mesh=mesh,
        scratch_types=[
            pltpu.VMEM((RING, CH), jnp.int32),
            pltpu.VMEM((RING * CH, H), _f32),
            pltpu.VMEM_SHARED((NPAD, H), _f32),
            [pltpu.SemaphoreType.DMA] * RING,
            [pltpu.SemaphoreType.DMA] * RING,
        ],
    )
    def k(msg_hbm, dst_hbm, init_hbm, out_hbm, idxr, bufm, acc, seml, sema):
        cid = lax.axis_index("c")
        sid = lax.axis_index("s")
        wid = sid * NC + cid
        row0 = sid * ROWS_PER_SUB
        pltpu.sync_copy(init_hbm.at[cid, pl.ds(row0, ROWS_PER_SUB)],
                        acc.at[pl.ds(row0, ROWS_PER_SUB)])
        plsc.subcore_barrier()

        def start_load(j, s):
            base = wid * EPW + j * CH
            pltpu.async_copy(msg_hbm.at[pl.ds(base, CH)],
                             bufm.at[pl.ds(s * CH, CH)], seml[s])
            pltpu.async_copy(dst_hbm.at[wid, j], idxr.at[s], seml[s])

        def drain_load(s):
            pltpu.make_async_copy(msg_hbm.at[pl.ds(0, CH)],
                                  bufm.at[pl.ds(s * CH, CH)], seml[s]).wait()
            pltpu.make_async_copy(dst_hbm.at[0, 0], idxr.at[s],
                                  seml[s]).wait()

        def drain_add(s):
            pltpu.make_async_copy(msg_hbm.at[pl.ds(0, CH)],
                                  bufm.at[pl.ds(s * CH, CH)], sema[s]).wait()

        def stage(j, s):
            s2 = (s + 2) % RING

            @pl.when(j <= NCHUNK - 3)
            def _():
                @pl.when(j >= 2)
                def _():
                    drain_add(s2)
                start_load(j + 2, s2)

            drain_load(s)
            pltpu.async_copy(bufm.at[pl.ds(s * CH, CH)], acc.at[idxr.at[s]],
                             sema[s], add=True)

        start_load(0, 0)
        start_load(1, 1)

        def body(jj, carry):
            for s in range(RING):
                stage(jj * RING + s, s)
            return carry

        lax.fori_loop(0, NCHUNK // RING, body, 0, unroll=False)
        stage(NCHUNK - 1, (NCHUNK - 1) % RING)
        for jt in range(NCHUNK - RING, NCHUNK):
            drain_add(jt % RING)
        plsc.subcore_barrier()
        pltpu.sync_copy(acc.at[pl.ds(row0, ROWS_PER_SUB)],
                        out_hbm.at[cid, pl.ds(row0, ROWS_PER_SUB)])

    return k(msg, dst, init)


def kernel(x, edge_attr, edge_index, params):
    src = edge_index[0]
    dst = edge_index[1]
    dst3c = [lax.slice(dst, (c * ECH,), ((c + 1) * ECH,)).reshape(
        NW, NCHUNK, CH) for c in range(K)]
    src3c = [lax.slice(src, (c * ECH,), ((c + 1) * ECH,)).reshape(
        NW, NCHUNK, CH) for c in range(K)]
    zeros_acc = jnp.zeros((NC, NPAD, H), _f32)

    def b2(v):
        return v.reshape(1, -1)

    def wb(w):
        return w.astype(_bf16)

    enc = params["node_enc"]
    eenc = params["edge_enc"]
    gnn = params["gnn"]
    dec = params["decoder"]

    h = _node_enc(x, enc["W0"], b2(enc["b0"]), enc["W1"], b2(enc["b1"]))
    e_chunks = [
        _edge_enc(lax.slice(edge_attr, (c * ECH, 0), ((c + 1) * ECH,
                                                      edge_attr.shape[1])),
                  eenc["W0"], b2(eenc["b0"]), wb(eenc["W1"]), b2(eenc["b1"]))
        for c in range(K)
    ]

    for l in range(2):
        p = gnn[l]
        em, nm = p["edge_mlp"], p["node_mlp"]
        wargs = (wb(em["W0"][0:H]), wb(em["W0"][H:2 * H]),
                 wb(nm["W0"][0:H]), wb(em["W0"][2 * H:3 * H]),
                 b2(em["b0"]), wb(em["W1"]), b2(em["b1"]),
                 wb(nm["W0"][H:2 * H]), b2(nm["b0"]),
                 wb(nm["W1"]), b2(nm["b1"]))
        gathered = [_sc_gather(h, dst3c[c], src3c[c]) for c in range(K)]
        s_parts = zeros_acc
        new_e = []
        for c in range(K):
            hd, hs = gathered[c]
            if l == 0:
                ec, msg = _edge_update(e_chunks[c], hd, hs, *wargs,
                                       b2(p["en_g"]), b2(p["en_b"]))
                new_e.append(ec)
            else:
                msg = _edge_update(e_chunks[c], hd, hs, *wargs, last=True)
            s_parts = _sc_scatter(msg, dst3c[c], s_parts)
        e_chunks = new_e
        if l == 0:
            h = _node_update(h, s_parts, b2(p["xn_g"]), b2(p["xn_b"]))
        else:
            out = _node_decode(h, s_parts, b2(p["xn_g"]), b2(p["xn_b"]),
                               dec["W0"], b2(dec["b0"]), dec["W1"],
                               b2(dec["b1"]))
    return out


# speedup vs baseline: 4.2661x; 1.1867x over previous
import functools

import jax
import jax.numpy as jnp
from jax import lax
from jax.experimental import pallas as pl
from jax.experimental.pallas import tpu as pltpu
from jax.experimental.pallas import tpu_sc as plsc

N = 10000
E = 320000
H = 128
OUT_DIM = 2

NC = 2
NS = 16
NW = NC * NS
CH = 80
ROWS_PER_SUB = 632
NPAD = NS * ROWS_PER_SUB

K = 5
ECH = E // K
EPW = ECH // NW
NCHUNK = EPW // CH

NB = 1000
EB = 4000

_f32 = jnp.float32
_bf16 = jnp.bfloat16


def _ln(x, g, b):
    m = jnp.mean(x, axis=-1, keepdims=True)
    v = jnp.mean((x - m) ** 2, axis=-1, keepdims=True)
    return (x - m) * lax.rsqrt(v + 1e-5) * g + b


def _dot(a, b):
    return jnp.dot(a, b, preferred_element_type=_f32)


def _bdot(a, b):
    return jnp.dot(a.astype(_bf16), b, preferred_element_type=_f32)


def _node_enc_body(x, w0, b0, w1, b1, h_o):
    h_o[...] = _dot(jax.nn.relu(_dot(x[...], w0[...]) + b0[...]), w1[...]) + b1[...]


def _node_enc(x, w0, b0, w1, b1):
    grid = N // NB
    blk = pl.BlockSpec((NB, H), lambda i: (i, 0))
    wspec = pl.BlockSpec((H, H), lambda i: (0, 0))
    bspec = pl.BlockSpec((1, H), lambda i: (0, 0))
    return pl.pallas_call(
        _node_enc_body,
        grid=(grid,),
        in_specs=[blk, wspec, bspec, wspec, bspec],
        out_specs=blk,
        out_shape=jax.ShapeDtypeStruct((N, H), _f32),
    )(x, w0, b0, w1, b1)


def _edge_core(e0, hdb, hsb, wa, wb, wd, wc, be0, we1, be1, wg, bn0, wn1,
               bn1):
    u = (_dot(hdb, wa[...]) + _dot(hsb, wb[...]) + _bdot(e0, wc[...])
         + be0[...])
    e_new = _bdot(jax.nn.relu(u), we1[...]) + be1[...]
    v = _dot(hdb, wd[...]) + _bdot(e_new, wg[...]) + bn0[...]
    msg = _bdot(jax.nn.relu(v), wn1[...]) + bn1[...]
    return e_new, msg


def _edge_body_l0(ea, hd, hs, ew0, eb0, ew1, eb1, wa, wb, wd, wc, be0, we1,
                  be1, wg, bn0, wn1, bn1, eng, enb, e_o, msg_o):
    e0 = _bdot(jax.nn.relu(_bdot(ea[...], ew0[...]) + eb0[...]), ew1[...]) \
        + eb1[...]
    hdb = hd[...].astype(_bf16)
    hsb = hs[...].astype(_bf16)
    e_new, msg = _edge_core(e0, hdb, hsb, wa, wb, wd, wc, be0, we1, be1, wg,
                            bn0, wn1, bn1)
    e_o[...] = _ln(e0 + e_new, eng[...], enb[...]).astype(_bf16)
    msg_o[...] = msg


def _edge_body_l1(e, hd, hs, wa, wb, wd, wc, be0, we1, be1, wg, bn0, wn1,
                  bn1, msg_o):
    hdb = hd[...].astype(_bf16)
    hsb = hs[...].astype(_bf16)
    _, msg = _edge_core(e[...].astype(_f32), hdb, hsb, wa, wb, wd, wc, be0,
                        we1, be1, wg, bn0, wn1, bn1)
    msg_o[...] = msg


def _edge_l0(c, ea_full, hd, hs, ew0, eb0, ew1, eb1, wa, wb, wd, wc, be0,
             we1, be1, wg, bn0, wn1, bn1, eng, enb):
    grid = ECH // EB
    d_edge = ea_full.shape[1]
    eablk = pl.BlockSpec((EB, d_edge), lambda i: (c * (ECH // EB) + i, 0))
    eblk = pl.BlockSpec((EB, H), lambda i: (i, 0))
    wspec = pl.BlockSpec((H, H), lambda i: (0, 0))
    bspec = pl.BlockSpec((1, H), lambda i: (0, 0))
    return pl.pallas_call(
        _edge_body_l0,
        grid=(grid,),
        in_specs=[eablk, eblk, eblk,
                  pl.BlockSpec((d_edge, H), lambda i: (0, 0)), bspec,
                  wspec, bspec,
                  wspec, wspec, wspec, wspec, bspec, wspec, bspec, wspec,
                  bspec, wspec, bspec, bspec, bspec],
        out_specs=[eblk, eblk],
        out_shape=[jax.ShapeDtypeStruct((ECH, H), _bf16),
                   jax.ShapeDtypeStruct((ECH, H), _f32)],
    )(ea_full, hd, hs, ew0, eb0, ew1, eb1, wa, wb, wd, wc, be0, we1, be1,
      wg, bn0, wn1, bn1, eng, enb)


def _edge_l1(e, hd, hs, wa, wb, wd, wc, be0, we1, be1, wg, bn0, wn1, bn1):
    grid = ECH // EB
    eblk = pl.BlockSpec((EB, H), lambda i: (i, 0))
    wspec = pl.BlockSpec((H, H), lambda i: (0, 0))
    bspec = pl.BlockSpec((1, H), lambda i: (0, 0))
    return pl.pallas_call(
        _edge_body_l1,
        grid=(grid,),
        in_specs=[eblk, eblk, eblk, wspec, wspec, wspec, wspec, bspec,
                  wspec, bspec, wspec, bspec, wspec, bspec],
        out_specs=eblk,
        out_shape=jax.ShapeDtypeStruct((ECH, H), _f32),
    )(e, hd, hs, wa, wb, wd, wc, be0, we1, be1, wg, bn0, wn1, bn1)


def _node_upd_body(h, s0, s1, g, b, h_o):
    h_o[...] = _ln(h[...] + s0[0] + s1[0], g[...], b[...])


def _node_update(h, s_parts, g, b):
    grid = N // NB
    blk = pl.BlockSpec((NB, H), lambda i: (i, 0))
    sblk = pl.BlockSpec((1, NB, H), lambda i: (0, i, 0))
    bspec = pl.BlockSpec((1, H), lambda i: (0, 0))
    return pl.pallas_call(
        _node_upd_body,
        grid=(grid,),
        in_specs=[blk, sblk, sblk, bspec, bspec],
        out_specs=blk,
        out_shape=jax.ShapeDtypeStruct((N, H), _f32),
    )(h, s_parts[0:1], s_parts[1:2], g, b)


def _node_dec_body(h, s0, s1, g, b, w0, b0, w1, b1, out_o):
    hn = _ln(h[...] + s0[0] + s1[0], g[...], b[...])
    out_o[...] = _dot(jax.nn.relu(_dot(hn, w0[...]) + b0[...]), w1[...]) + b1[...]


def _node_decode(h, s_parts, g, b, w0, b0, w1, b1):
    grid = N // NB
    blk = pl.BlockSpec((NB, H), lambda i: (i, 0))
    sblk = pl.BlockSpec((1, NB, H), lambda i: (0, i, 0))
    wspec = pl.BlockSpec((H, H), lambda i: (0, 0))
    bspec = pl.BlockSpec((1, H), lambda i: (0, 0))
    return pl.pallas_call(
        _node_dec_body,
        grid=(grid,),
        in_specs=[blk, sblk, sblk, bspec, bspec, wspec, bspec,
                  pl.BlockSpec((H, OUT_DIM), lambda i: (0, 0)),
                  pl.BlockSpec((1, OUT_DIM), lambda i: (0, 0))],
        out_specs=pl.BlockSpec((NB, OUT_DIM), lambda i: (i, 0)),
        out_shape=jax.ShapeDtypeStruct((N, OUT_DIM), _f32),
    )(h, s_parts[0:1], s_parts[1:2], g, b, w0, b0, w1, b1)


RING = 4


def _sc_gather(h, dst3, src3):
    mesh = plsc.VectorSubcoreMesh(core_axis_name="c", subcore_axis_name="s")

    @functools.partial(
        pl.kernel,
        out_type=[jax.ShapeDtypeStruct((ECH, H), _f32)] * 2,
        mesh=mesh,
        scratch_types=[
            pltpu.VMEM((NCHUNK, CH), jnp.int32),
            pltpu.VMEM((NCHUNK, CH), jnp.int32),
            pltpu.VMEM((RING * CH, H), _f32),
            pltpu.VMEM((RING * CH, H), _f32),
            [pltpu.SemaphoreType.DMA] * RING,
            [pltpu.SemaphoreType.DMA] * RING,
        ],
    )
    def k(h_hbm, dst_hbm, src_hbm, hd_hbm, hs_hbm, dsti, srci, bufd, bufs,
          semg, semw):
        wid = lax.axis_index("s") * NC + lax.axis_index("c")
        pltpu.sync_copy(dst_hbm.at[wid], dsti)
        pltpu.sync_copy(src_hbm.at[wid], srci)

        def start_gather(j, s):
            off = s * CH
            pltpu.async_copy(h_hbm.at[dsti.at[j]], bufd.at[pl.ds(off, CH)],
                             semg[s])
            pltpu.async_copy(h_hbm.at[srci.at[j]], bufs.at[pl.ds(off, CH)],
                             semg[s])

        def drain_gather(s):
            off = s * CH
            pltpu.make_async_copy(hd_hbm.at[pl.ds(0, CH)],
                                  bufd.at[pl.ds(off, CH)], semg[s]).wait()
            pltpu.make_async_copy(hd_hbm.at[pl.ds(0, CH)],
                                  bufs.at[pl.ds(off, CH)], semg[s]).wait()

        def start_writeback(j, s):
            off = s * CH
            base = wid * EPW + j * CH
            pltpu.async_copy(bufd.at[pl.ds(off, CH)],
                             hd_hbm.at[pl.ds(base, CH)], semw[s])
            pltpu.async_copy(bufs.at[pl.ds(off, CH)],
                             hs_hbm.at[pl.ds(base, CH)], semw[s])

        def drain_writeback(s):
            off = s * CH
            pltpu.make_async_copy(hd_hbm.at[pl.ds(0, CH)],
                                  bufd.at[pl.ds(off, CH)], semw[s]).wait()
            pltpu.make_async_copy(hd_hbm.at[pl.ds(0, CH)],
                                  bufs.at[pl.ds(off, CH)], semw[s]).wait()

        def stage(j, s):
            s2 = (s + 2) % RING

            @pl.when(j <= NCHUNK - 3)
            def _():
                @pl.when(j >= 2)
                def _():
                    drain_writeback(s2)
                start_gather(j + 2, s2)

            drain_gather(s)
            start_writeback(j, s)

        start_gather(0, 0)
        start_gather(1, 1)

        def body(jj, carry):
            for s in range(RING):
                stage(jj * RING + s, s)
            return carry

        lax.fori_loop(0, NCHUNK // RING, body, 0, unroll=False)
        stage(NCHUNK - 1, (NCHUNK - 1) % RING)
        for jt in range(NCHUNK - RING, NCHUNK):
            drain_writeback(jt % RING)

    return k(h, dst3, src3)


def _sc_scatter(msg, dst, init):
    mesh = plsc.VectorSubcoreMesh(core_axis_name="c", subcore_axis_name="s")

    @functools.partial(
        pl.kernel,
        out_type=jax.ShapeDtypeStruct((NC, NPAD, H), _f32),
        mesh=mesh,
        scratch_types=[
            pltpu.VMEM((RING, CH), jnp.int32),
            pltpu.VMEM((RING * CH, H), _f32),
            pltpu.VMEM_SHARED((NPAD, H), _f32),
            [pltpu.SemaphoreType.DMA] * RING,
            [pltpu.SemaphoreType.DMA] * RING,
        ],
    )
    def k(msg_hbm, dst_hbm, init_hbm, out_hbm, idxr, bufm, acc, seml, sema):
        cid = lax.axis_index("c")
        sid = lax.axis_index("s")
        wid = sid * NC + cid
        row0 = sid * ROWS_PER_SUB
        pltpu.sync_copy(init_hbm.at[cid, pl.ds(row0, ROWS_PER_SUB)],
                        acc.at[pl.ds(row0, ROWS_PER_SUB)])
        plsc.subcore_barrier()

        def start_load(j, s):
            base = wid * EPW + j * CH
            pltpu.async_copy(msg_hbm.at[pl.ds(base, CH)],
                             bufm.at[pl.ds(s * CH, CH)], seml[s])
            pltpu.async_copy(dst_hbm.at[wid, j], idxr.at[s], seml[s])

        def drain_load(s):
            pltpu.make_async_copy(msg_hbm.at[pl.ds(0, CH)],
                                  bufm.at[pl.ds(s * CH, CH)], seml[s]).wait()
            pltpu.make_async_copy(dst_hbm.at[0, 0], idxr.at[s],
                                  seml[s]).wait()

        def drain_add(s):
            pltpu.make_async_copy(msg_hbm.at[pl.ds(0, CH)],
                                  bufm.at[pl.ds(s * CH, CH)], sema[s]).wait()

        def stage(j, s):
            s2 = (s + 2) % RING

            @pl.when(j <= NCHUNK - 3)
            def _():
                @pl.when(j >= 2)
                def _():
                    drain_add(s2)
                start_load(j + 2, s2)

            drain_load(s)
            pltpu.async_copy(bufm.at[pl.ds(s * CH, CH)], acc.at[idxr.at[s]],
                             sema[s], add=True)

        start_load(0, 0)
        start_load(1, 1)

        def body(jj, carry):
            for s in range(RING):
                stage(jj * RING + s, s)
            return carry

        lax.fori_loop(0, NCHUNK // RING, body, 0, unroll=False)
        stage(NCHUNK - 1, (NCHUNK - 1) % RING)
        for jt in range(NCHUNK - RING, NCHUNK):
            drain_add(jt % RING)
        plsc.subcore_barrier()
        pltpu.sync_copy(acc.at[pl.ds(row0, ROWS_PER_SUB)],
                        out_hbm.at[cid, pl.ds(row0, ROWS_PER_SUB)])

    return k(msg, dst, init)


def kernel(x, edge_attr, edge_index, params):
    src = edge_index[0]
    dst = edge_index[1]
    dst3c = [lax.slice(dst, (c * ECH,), ((c + 1) * ECH,)).reshape(
        NW, NCHUNK, CH) for c in range(K)]
    src3c = [lax.slice(src, (c * ECH,), ((c + 1) * ECH,)).reshape(
        NW, NCHUNK, CH) for c in range(K)]
    zeros_acc = jnp.zeros((NC, NPAD, H), _f32)

    def b2(v):
        return v.reshape(1, -1)

    def wb(w):
        return w.astype(_bf16)

    enc = params["node_enc"]
    eenc = params["edge_enc"]
    gnn = params["gnn"]
    dec = params["decoder"]

    h = _node_enc(x, enc["W0"], b2(enc["b0"]), enc["W1"], b2(enc["b1"]))

    e_chunks = []
    for l in range(2):
        p = gnn[l]
        em, nm = p["edge_mlp"], p["node_mlp"]
        wargs = (wb(em["W0"][0:H]), wb(em["W0"][H:2 * H]),
                 wb(nm["W0"][0:H]), wb(em["W0"][2 * H:3 * H]),
                 b2(em["b0"]), wb(em["W1"]), b2(em["b1"]),
                 wb(nm["W0"][H:2 * H]), b2(nm["b0"]),
                 wb(nm["W1"]), b2(nm["b1"]))
        gathered = [_sc_gather(h, dst3c[c], src3c[c]) for c in range(K)]
        s_parts = zeros_acc
        for c in range(K):
            hd, hs = gathered[c]
            if l == 0:
                ec, msg = _edge_l0(
                    c, edge_attr, hd, hs,
                    wb(eenc["W0"]), b2(eenc["b0"]), wb(eenc["W1"]),
                    b2(eenc["b1"]), *wargs, b2(p["en_g"]), b2(p["en_b"]))
                e_chunks.append(ec)
            else:
                msg = _edge_l1(e_chunks[c], hd, hs, *wargs)
            s_parts = _sc_scatter(msg, dst3c[c], s_parts)
        if l == 0:
            h = _node_update(h, s_parts, b2(p["xn_g"]), b2(p["xn_b"]))
        else:
            out = _node_decode(h, s_parts, b2(p["xn_g"]), b2(p["xn_b"]),
                               dec["W0"], b2(dec["b0"]), dec["W1"],
                               b2(dec["b1"]))
    return out
